# baseline (speedup 1.0000x reference)
import functools

import jax
import jax.numpy as jnp
from jax import lax
from jax.experimental import pallas as pl
from jax.experimental.pallas import tpu as pltpu
from jax.experimental.pallas import tpu_sc as plsc

_PLANES = 64
_VOCAB = 23
_BATCH = 4096
_L = 200
_LANES = 16

_INFO = plsc.get_sparse_core_info()
_NC = _INFO.num_cores
_NS = _INFO.num_subcores
_NW = _NC * _NS
_ROWS = _BATCH // _NW

_CHUNK_BASES = tuple(16 * j for j in range(_L // _LANES)) + (_L - _LANES,)


def _sc_body(x_hbm, wt_hbm, out_hbm, x_v, wt_v, out_buf, sem0, sem1):
    wid = lax.axis_index("s") * _NC + lax.axis_index("c")
    base_row = wid * _ROWS
    pltpu.sync_copy(wt_hbm, wt_v)
    pltpu.sync_copy(x_hbm.at[pl.ds(base_row, _ROWS)], x_v)
    sems = (sem0, sem1)

    def compute_row(k, r_local):
        lane = lax.iota(jnp.int32, _LANES)
        xv = [
            x_v[r_local, pl.ds(b, _LANES)] * _LANES + lane for b in _CHUNK_BASES
        ]

        @plsc.parallel_loop(0, _PLANES, step=2, unroll=4)
        def c_body(c):
            vals = []
            for g in range(2):
                coff = (c + g) * (_VOCAB * _LANES)
                vals.append(
                    [plsc.load_gather(wt_v, [xv[j] + coff]) for j in range(13)]
                )
            for g in range(2):
                for j, b in enumerate(_CHUNK_BASES):
                    out_buf[k, c + g, pl.ds(b, _LANES)] = vals[g][j]

    DIAG_NO_DMA = False

    def pair_body(r2, carry):
        for k in range(2):
            r = r2 * 2 + k

            if not DIAG_NO_DMA:

                @pl.when(r2 > 0)
                def _wait_prev():
                    pltpu.make_async_copy(
                        out_buf.at[k], out_hbm.at[base_row + r - 2], sems[k]
                    ).wait()

            compute_row(k, r)
            if not DIAG_NO_DMA:
                pltpu.make_async_copy(
                    out_buf.at[k], out_hbm.at[base_row + r], sems[k]
                ).start()
        return carry

    lax.fori_loop(0, _ROWS // 2, pair_body, 0)
    for k in range(2):
        if DIAG_NO_DMA:
            pltpu.make_async_copy(
                out_buf.at[k], out_hbm.at[base_row + _ROWS - 2 + k], sems[k]
            ).start()
        pltpu.make_async_copy(
            out_buf.at[k], out_hbm.at[base_row + _ROWS - 2 + k], sems[k]
        ).wait()


_sc_call = functools.partial(
    pl.kernel,
    out_type=jax.ShapeDtypeStruct((_BATCH, _PLANES, _L), jnp.float32),
    mesh=plsc.VectorSubcoreMesh(core_axis_name="c", subcore_axis_name="s"),
    scratch_types=[
        pltpu.VMEM((_ROWS, _L), jnp.int32),
        pltpu.VMEM((_PLANES * _VOCAB * _LANES,), jnp.float32),
        pltpu.VMEM((2, _PLANES, _L), jnp.float32),
        pltpu.SemaphoreType.DMA,
        pltpu.SemaphoreType.DMA,
    ],
    compiler_params=pltpu.CompilerParams(
        needs_layout_passes=False, use_tc_tiling_on_sc=False
    ),
)(_sc_body)


@jax.jit
def kernel(x, emb_weight):
    wt_flat = jnp.transpose(emb_weight).reshape(-1)
    wt_rep = jnp.broadcast_to(wt_flat[:, None], (wt_flat.shape[0], _LANES))
    return _sc_call(x, wt_rep.reshape(-1))


# speedup vs baseline: 3.1181x; 3.1181x over previous
import functools

import jax
import jax.numpy as jnp
from jax import lax
from jax.experimental import pallas as pl
from jax.experimental.pallas import tpu as pltpu
from jax.experimental.pallas import tpu_sc as plsc

_PLANES = 64
_VOCAB = 23
_BATCH = 4096
_L = 200
_LANES = 16

_INFO = plsc.get_sparse_core_info()
_NC = _INFO.num_cores
_NS = _INFO.num_subcores
_NW = _NC * _NS
_ROWS = _BATCH // _NW

_CHUNK_BASES = tuple(16 * j for j in range(_L // _LANES)) + (_L - _LANES,)


def _sc_body(x_hbm, wt_hbm, out_hbm, x_v, wt_v, out_buf, sem0, sem1):
    wid = lax.axis_index("s") * _NC + lax.axis_index("c")
    base_row = wid * _ROWS
    pltpu.sync_copy(wt_hbm, wt_v)
    pltpu.sync_copy(x_hbm.at[pl.ds(base_row, _ROWS)], x_v)
    sems = (sem0, sem1)

    def compute_row(k, r_local):
        lane = lax.iota(jnp.int32, _LANES)
        xv = [
            x_v[r_local, pl.ds(b, _LANES)] * _LANES + lane for b in _CHUNK_BASES
        ]

        @plsc.parallel_loop(0, _PLANES, step=8)
        def c_body(c):
            c8 = pl.multiple_of(c, 8)
            for g in range(8):
                coff = c8 * (_VOCAB * _LANES) + g * (_VOCAB * _LANES)
                vals = [
                    plsc.load_gather(wt_v, [xv[j] + coff]) for j in range(13)
                ]
                for j, b in enumerate(_CHUNK_BASES):
                    out_buf[k, c8 + g, pl.ds(b, _LANES)] = vals[j]

    DIAG_NO_DMA = False

    def pair_body(r2, carry):
        for k in range(2):
            r = r2 * 2 + k

            if not DIAG_NO_DMA:

                @pl.when(r2 > 0)
                def _wait_prev():
                    pltpu.make_async_copy(
                        out_buf.at[k], out_hbm.at[base_row + r - 2], sems[k]
                    ).wait()

            compute_row(k, r)
            if not DIAG_NO_DMA:
                pltpu.make_async_copy(
                    out_buf.at[k], out_hbm.at[base_row + r], sems[k]
                ).start()
        return carry

    lax.fori_loop(0, _ROWS // 2, pair_body, 0)
    for k in range(2):
        if DIAG_NO_DMA:
            pltpu.make_async_copy(
                out_buf.at[k], out_hbm.at[base_row + _ROWS - 2 + k], sems[k]
            ).start()
        pltpu.make_async_copy(
            out_buf.at[k], out_hbm.at[base_row + _ROWS - 2 + k], sems[k]
        ).wait()


_sc_call = functools.partial(
    pl.kernel,
    out_type=jax.ShapeDtypeStruct((_BATCH, _PLANES, _L), jnp.float32),
    mesh=plsc.VectorSubcoreMesh(core_axis_name="c", subcore_axis_name="s"),
    scratch_types=[
        pltpu.VMEM((_ROWS, _L), jnp.int32),
        pltpu.VMEM((_PLANES * _VOCAB * _LANES,), jnp.float32),
        pltpu.VMEM((2, _PLANES, _L), jnp.float32),
        pltpu.SemaphoreType.DMA,
        pltpu.SemaphoreType.DMA,
    ],
    compiler_params=pltpu.CompilerParams(needs_layout_passes=False),
)(_sc_body)


@jax.jit
def kernel(x, emb_weight):
    wt_flat = jnp.transpose(emb_weight).reshape(-1)
    wt_rep = jnp.broadcast_to(wt_flat[:, None], (wt_flat.shape[0], _LANES))
    return _sc_call(x, wt_rep.reshape(-1))


# speedup vs baseline: 3.1194x; 1.0004x over previous
import functools

import jax
import jax.numpy as jnp
from jax import lax
from jax.experimental import pallas as pl
from jax.experimental.pallas import tpu as pltpu
from jax.experimental.pallas import tpu_sc as plsc

_PLANES = 64
_VOCAB = 23
_BATCH = 4096
_L = 200
_LANES = 16

_INFO = plsc.get_sparse_core_info()
_NC = _INFO.num_cores
_NS = _INFO.num_subcores
_NW = _NC * _NS
_ROWS = _BATCH // _NW

_CHUNK_BASES = tuple(16 * j for j in range(_L // _LANES)) + (_L - _LANES,)


def _sc_body(x_hbm, wt_hbm, out_hbm, x_v, wt_v, out_buf, sem0, sem1):
    wid = lax.axis_index("s") * _NC + lax.axis_index("c")
    base_row = wid * _ROWS
    pltpu.sync_copy(wt_hbm, wt_v)
    pltpu.sync_copy(x_hbm.at[pl.ds(base_row, _ROWS)], x_v)
    sems = (sem0, sem1)

    def compute_row(k, r_local):
        lane = lax.iota(jnp.int32, _LANES)
        xv = [
            x_v[r_local, pl.ds(b, _LANES)] * _LANES + lane for b in _CHUNK_BASES
        ]

        @plsc.parallel_loop(0, _PLANES, step=8)
        def c_body(c):
            c8 = pl.multiple_of(c, 8)
            for g in range(8):
                coff = c8 * (_VOCAB * _LANES) + g * (_VOCAB * _LANES)
                vals = [
                    plsc.load_gather(wt_v, [xv[j] + coff]) for j in range(13)
                ]
                for j, b in enumerate(_CHUNK_BASES):
                    out_buf[k, c8 + g, pl.ds(b, _LANES)] = vals[j]

    DIAG_NO_DMA = False

    def pair_body(r2, carry):
        for k in range(2):
            r = r2 * 2 + k

            if not DIAG_NO_DMA:

                @pl.when(r2 > 0)
                def _wait_prev():
                    pltpu.make_async_copy(
                        out_buf.at[k], out_hbm.at[base_row + r - 2], sems[k]
                    ).wait()

            compute_row(k, r)
            if not DIAG_NO_DMA:
                pltpu.make_async_copy(
                    out_buf.at[k], out_hbm.at[base_row + r], sems[k]
                ).start()
        return carry

    lax.fori_loop(0, _ROWS // 2, pair_body, 0)
    for k in range(2):
        if DIAG_NO_DMA:
            pltpu.make_async_copy(
                out_buf.at[k], out_hbm.at[base_row + _ROWS - 2 + k], sems[k]
            ).start()
        pltpu.make_async_copy(
            out_buf.at[k], out_hbm.at[base_row + _ROWS - 2 + k], sems[k]
        ).wait()


_sc_call = functools.partial(
    pl.kernel,
    out_type=jax.ShapeDtypeStruct((_BATCH, _PLANES, _L), jnp.float32),
    mesh=plsc.VectorSubcoreMesh(core_axis_name="c", subcore_axis_name="s"),
    scratch_types=[
        pltpu.VMEM((_ROWS, _L), jnp.int32),
        pltpu.VMEM((_PLANES * _VOCAB * _LANES,), jnp.float32),
        pltpu.VMEM((2, _PLANES, _L), jnp.float32),
        pltpu.SemaphoreType.DMA,
        pltpu.SemaphoreType.DMA,
    ],
    compiler_params=pltpu.CompilerParams(
        needs_layout_passes=False, use_tc_tiling_on_sc=True
    ),
)(_sc_body)


@jax.jit
def kernel(x, emb_weight):
    wt_flat = jnp.transpose(emb_weight).reshape(-1)
    wt_rep = jnp.broadcast_to(wt_flat[:, None], (wt_flat.shape[0], _LANES))
    return _sc_call(x, wt_rep.reshape(-1))


# speedup vs baseline: 8.5727x; 2.7482x over previous
import functools

import jax
import jax.numpy as jnp
from jax import lax
from jax.experimental import pallas as pl
from jax.experimental.pallas import tpu as pltpu
from jax.experimental.pallas import tpu_sc as plsc

_PLANES = 64
_VOCAB = 23
_BATCH = 4096
_L = 200
_LANES = 16

_INFO = plsc.get_sparse_core_info()
_NC = _INFO.num_cores
_NS = _INFO.num_subcores
_NW = _NC * _NS
_BW = _BATCH // _NW
_LB = 2
_NBLK = _L // _LB
_JCH = _BW // _LANES


def _sc_body(xt_hbm, wt_hbm, out_hbm, x_v, wt_v, out_blk, sem0, sem1):
    wid = lax.axis_index("s") * _NC + lax.axis_index("c")
    b0 = wid * _BW
    pltpu.sync_copy(wt_hbm, wt_v)
    pltpu.sync_copy(xt_hbm.at[:, pl.ds(b0, _BW)], x_v)
    sems = (sem0, sem1)
    lane = lax.iota(jnp.int32, _LANES)

    def compute_block(k, l0):
        xv = [
            x_v[l0 + dl, pl.ds(_LANES * j, _LANES)] * _LANES + lane
            for dl in range(_LB)
            for j in range(_JCH)
        ]

        @plsc.parallel_loop(0, _PLANES, step=8)
        def c_body(c):
            c8 = pl.multiple_of(c, 8)
            for g in range(8):
                coff = c8 * (_VOCAB * _LANES) + g * (_VOCAB * _LANES)
                vals = [
                    plsc.load_gather(wt_v, [xv[u] + coff])
                    for u in range(_LB * _JCH)
                ]
                for u in range(_LB * _JCH):
                    out_blk[k, c8 + g, pl.ds(_LANES * u, _LANES)] = vals[u]

    def pair_body(blk2, carry):
        for k in range(2):
            blk = blk2 * 2 + k
            l0 = blk * _LB

            @pl.when(blk2 > 0)
            def _wait_prev():
                for dl in range(_LB):
                    pltpu.make_async_copy(
                        out_blk.at[k, :, pl.ds(128 * dl, 128)],
                        out_hbm.at[:, l0 - 2 * _LB + dl, pl.ds(b0, _BW)],
                        sems[k],
                    ).wait()

            compute_block(k, l0)
            for dl in range(_LB):
                pltpu.make_async_copy(
                    out_blk.at[k, :, pl.ds(128 * dl, 128)],
                    out_hbm.at[:, l0 + dl, pl.ds(b0, _BW)],
                    sems[k],
                ).start()
        return carry

    lax.fori_loop(0, _NBLK // 2, pair_body, 0)
    for k in range(2):
        l0 = (_NBLK - 2 + k) * _LB
        for dl in range(_LB):
            pltpu.make_async_copy(
                out_blk.at[k, :, pl.ds(128 * dl, 128)],
                out_hbm.at[:, l0 + dl, pl.ds(b0, _BW)],
                sems[k],
            ).wait()


_sc_call = functools.partial(
    pl.kernel,
    out_type=jax.ShapeDtypeStruct((_PLANES, _L, _BATCH), jnp.float32),
    mesh=plsc.VectorSubcoreMesh(core_axis_name="c", subcore_axis_name="s"),
    scratch_types=[
        pltpu.VMEM((_L, _BW), jnp.int32),
        pltpu.VMEM((_PLANES * _VOCAB * _LANES,), jnp.float32),
        pltpu.VMEM((2, _PLANES, _LB * 128), jnp.float32),
        pltpu.SemaphoreType.DMA,
        pltpu.SemaphoreType.DMA,
    ],
    compiler_params=pltpu.CompilerParams(
        needs_layout_passes=False, use_tc_tiling_on_sc=True
    ),
)(_sc_body)


@jax.jit
def kernel(x, emb_weight):
    wt_flat = jnp.transpose(emb_weight).reshape(-1)
    wt_rep = jnp.broadcast_to(wt_flat[:, None], (wt_flat.shape[0], _LANES))
    out_clb = _sc_call(jnp.transpose(x), wt_rep.reshape(-1))
    return jnp.transpose(out_clb, (2, 0, 1))
